# Initial kernel scaffold; baseline (speedup 1.0000x reference)
import jax
import jax.numpy as jnp
from jax.experimental import pallas as pl


def kernel(idx, word_emb, pos_emb):
    raise NotImplementedError("write your pallas kernel here")


# speedup vs baseline: 1.1424x; 1.1424x over previous
import functools

import jax
import jax.numpy as jnp
from jax import lax
from jax.experimental import pallas as pl
from jax.experimental.pallas import tpu as pltpu
from jax.experimental.pallas import tpu_sc as plsc

_LANES = 16


def _emb_lookup(idx, word_emb, pos_emb, num_cores, num_subcores):
    B, T = idx.shape
    V, D = word_emb.shape
    NW = num_cores * num_subcores
    CH = T // NW

    mesh = plsc.VectorSubcoreMesh(core_axis_name="c", subcore_axis_name="s")

    @functools.partial(
        pl.kernel,
        mesh=mesh,
        out_type=jax.ShapeDtypeStruct((B, T, D), jnp.float32),
        scratch_types=[
            pltpu.VMEM((CH,), jnp.int32),
            pltpu.VMEM((CH, D), jnp.float32),
            pltpu.VMEM((CH, D), jnp.float32),
            pltpu.SemaphoreType.DMA,
        ],
    )
    def emb_kernel(idx_hbm, word_hbm, pos_hbm, out_hbm, idx_v, pos_v, rows_v, sem):
        wid = lax.axis_index("s") * num_cores + lax.axis_index("c")
        t0 = wid * CH
        pltpu.sync_copy(pos_hbm.at[pl.ds(t0, CH)], pos_v)
        for b in range(B):
            pltpu.sync_copy(idx_hbm.at[b, pl.ds(t0, CH)], idx_v)
            pltpu.async_copy(word_hbm.at[idx_v], rows_v, sem).wait()

            def add_row(i, carry):
                for k in range(D // _LANES):
                    sl = pl.ds(k * _LANES, _LANES)
                    plsc.addupdate(rows_v.at[i, sl], pos_v[i, sl])
                return carry

            lax.fori_loop(0, CH, add_row, 0)
            pltpu.sync_copy(rows_v, out_hbm.at[b, pl.ds(t0, CH)])

    return emb_kernel(idx, word_emb, pos_emb)


def kernel(idx, word_emb, pos_emb):
    idx = jnp.asarray(idx, jnp.int32)
    return _emb_lookup(idx, word_emb, pos_emb, num_cores=2, num_subcores=16)
